# Initial kernel scaffold; baseline (speedup 1.0000x reference)
import jax
import jax.numpy as jnp
from jax.experimental import pallas as pl


def kernel(x, codebook):
    raise NotImplementedError("write your pallas kernel here")


# speedup vs baseline: 1.1585x; 1.1585x over previous
import jax
import jax.numpy as jnp
from jax.experimental import pallas as pl
from jax.experimental.pallas import tpu as pltpu
from jax.experimental.pallas import tpu_sc as plsc

N_EMB = 8192
DIM = 256
M_TOTAL = 8192
M_BLK = 128
NB = M_TOTAL // M_BLK

GATHER_WIN = 128


def _tc_body(x_ref, cb_ref, idx_ref, oh_ref, csq_ref, previdx_ref):
    i = pl.program_id(0)
    ids = jax.lax.broadcasted_iota(jnp.int32, (M_BLK, N_EMB), 1)

    oh_ref[...] = (ids == previdx_ref[...]).astype(jnp.float32)

    @pl.when(i == 0)
    def _csq():
        cb = cb_ref[...]
        ones = jnp.ones((1, DIM), jnp.float32)
        csq_ref[...] = jax.lax.dot_general(
            ones, cb * cb, (((1,), (1,)), ((), ())),
            precision=jax.lax.Precision.HIGHEST,
            preferred_element_type=jnp.float32)

    @pl.when(i < NB)
    def _compute():
        xb = x_ref[...]
        cb = cb_ref[...]
        dot = jax.lax.dot_general(
            xb * -2.0, cb, (((1,), (1,)), ((), ())),
            preferred_element_type=jnp.float32)
        xsq = jnp.sum(xb * xb, axis=1, keepdims=True)
        scores = (xsq + dot) + csq_ref[...]
        mn = jnp.min(scores, axis=1, keepdims=True)
        idx = jnp.min(jnp.where(scores == mn, ids, jnp.int32(N_EMB)),
                      axis=1, keepdims=True)
        idx_ref[0, :, :] = idx
        previdx_ref[...] = idx

    @pl.when(i == NB)
    def _tail():
        idx_ref[0, :, :] = previdx_ref[...]


def _tc_argmin_onehot(x_flat, codebook, interpret=False):
    grid = (NB + 1,)
    out_shapes = (
        jax.ShapeDtypeStruct((NB, M_BLK, 1), jnp.int32),
        jax.ShapeDtypeStruct((M_TOTAL, N_EMB), jnp.float32),
    )
    return pl.pallas_call(
        _tc_body,
        grid=grid,
        in_specs=[
            pl.BlockSpec((M_BLK, DIM), lambda i: (jnp.minimum(i, NB - 1), 0)),
            pl.BlockSpec((N_EMB, DIM), lambda i: (0, 0)),
        ],
        out_specs=(
            pl.BlockSpec((1, M_BLK, 1), lambda i: (jnp.minimum(i, NB - 1), 0, 0)),
            pl.BlockSpec((M_BLK, N_EMB), lambda i: (jnp.maximum(i - 1, 0), 0)),
        ),
        out_shape=out_shapes,
        scratch_shapes=[
            pltpu.VMEM((1, N_EMB), jnp.float32),
            pltpu.VMEM((M_BLK, 1), jnp.int32),
        ],
        interpret=interpret,
    )(x_flat, codebook)


def _sc_gather(codebook, idx_row):
    mesh = plsc.VectorSubcoreMesh(core_axis_name="core",
                                  subcore_axis_name="subcore")

    @pl.kernel(out_type=jax.ShapeDtypeStruct((M_TOTAL, DIM), jnp.float32),
               mesh=mesh)
    def knl(cb_hbm, i_hbm, o_hbm):
        def body(i_vmem, o_vmem):
            pltpu.sync_copy(cb_hbm.at[i_vmem.at[0]], o_vmem)

        pltpu.emit_pipeline(
            body,
            grid=(M_TOTAL // GATHER_WIN,),
            in_specs=[pl.BlockSpec((1, GATHER_WIN), lambda i: (0, i))],
            out_specs=[pl.BlockSpec((GATHER_WIN, DIM), lambda i: (i, 0))],
            core_axis_name=("core", "subcore"),
            dimension_semantics=(pltpu.PARALLEL,),
        )(i_hbm, o_hbm)

    return knl(codebook, idx_row)


def kernel(x, codebook):
    x_flat = x.reshape(-1, DIM)
    idx_blocks, discrete = _tc_argmin_onehot(x_flat, codebook)
    idx_row = idx_blocks.reshape(1, M_TOTAL)
    quantized = _sc_gather(codebook, idx_row).reshape(x.shape)
    return (discrete, quantized)


# speedup vs baseline: 1.5517x; 1.3394x over previous
import jax
import jax.numpy as jnp
from jax.experimental import pallas as pl
from jax.experimental.pallas import tpu as pltpu
from jax.experimental.pallas import tpu_sc as plsc

N_EMB = 8192
DIM = 256
M_TOTAL = 8192
M_BLK = 256
NB = M_TOTAL // M_BLK
N_CHUNK = 512
NC = N_EMB // N_CHUNK

GATHER_WIN = 128


def _tc_body(x_ref, cb_ref, idx_ref, oh_ref, csq_ref, cbbf_ref, previdx_ref):
    i = pl.program_id(0)

    @pl.when(i == 0)
    def _init():
        cb = cb_ref[...]
        ones = jnp.ones((1, DIM), jnp.float32)
        csq_ref[...] = jax.lax.dot_general(
            ones, cb * cb, (((1,), (1,)), ((), ())),
            precision=jax.lax.Precision.HIGHEST,
            preferred_element_type=jnp.float32)
        cbbf_ref[...] = cb.astype(jnp.bfloat16)

    ids = jax.lax.broadcasted_iota(jnp.int32, (M_BLK, N_EMB), 1)
    oh_ref[...] = (ids == previdx_ref[...]).astype(jnp.float32)

    @pl.when(i < NB)
    def _compute():
        xb = x_ref[...]
        xsq = jnp.sum(xb * xb, axis=1, keepdims=True)
        xbf = (xb * -2.0).astype(jnp.bfloat16)
        iota_w = jax.lax.broadcasted_iota(jnp.int32, (M_BLK, N_CHUNK), 1)
        minv = None
        argv = None
        for k in range(NC):
            cbk = cbbf_ref[pl.ds(k * N_CHUNK, N_CHUNK), :]
            dot = jax.lax.dot_general(
                xbf, cbk, (((1,), (1,)), ((), ())),
                preferred_element_type=jnp.float32)
            csqk = csq_ref[:, pl.ds(k * N_CHUNK, N_CHUNK)]
            d = (xsq + dot) + csqk
            col = iota_w + (k * N_CHUNK)
            if minv is None:
                minv, argv = d, col
            else:
                upd = d < minv
                minv = jnp.where(upd, d, minv)
                argv = jnp.where(upd, col, argv)
        mn = jnp.min(minv, axis=1, keepdims=True)
        idx = jnp.min(jnp.where(minv == mn, argv, jnp.int32(N_EMB)),
                      axis=1, keepdims=True)
        idx_ref[0, :, :] = idx
        previdx_ref[...] = idx

    @pl.when(i == NB)
    def _tail():
        idx_ref[0, :, :] = previdx_ref[...]


def _tc_argmin_onehot(x_flat, codebook, interpret=False):
    grid = (NB + 1,)
    out_shapes = (
        jax.ShapeDtypeStruct((NB, M_BLK, 1), jnp.int32),
        jax.ShapeDtypeStruct((M_TOTAL, N_EMB), jnp.float32),
    )
    return pl.pallas_call(
        _tc_body,
        grid=grid,
        in_specs=[
            pl.BlockSpec((M_BLK, DIM), lambda i: (jnp.minimum(i, NB - 1), 0)),
            pl.BlockSpec((N_EMB, DIM), lambda i: (0, 0)),
        ],
        out_specs=(
            pl.BlockSpec((1, M_BLK, 1), lambda i: (jnp.minimum(i, NB - 1), 0, 0)),
            pl.BlockSpec((M_BLK, N_EMB), lambda i: (jnp.maximum(i - 1, 0), 0)),
        ),
        out_shape=out_shapes,
        scratch_shapes=[
            pltpu.VMEM((1, N_EMB), jnp.float32),
            pltpu.VMEM((N_EMB, DIM), jnp.bfloat16),
            pltpu.VMEM((M_BLK, 1), jnp.int32),
        ],
        interpret=interpret,
    )(x_flat, codebook)


def _sc_gather(codebook, idx_row):
    mesh = plsc.VectorSubcoreMesh(core_axis_name="core",
                                  subcore_axis_name="subcore")

    @pl.kernel(out_type=jax.ShapeDtypeStruct((M_TOTAL, DIM), jnp.float32),
               mesh=mesh)
    def knl(cb_hbm, i_hbm, o_hbm):
        def body(i_vmem, o_vmem):
            pltpu.sync_copy(cb_hbm.at[i_vmem.at[0]], o_vmem)

        pltpu.emit_pipeline(
            body,
            grid=(M_TOTAL // GATHER_WIN,),
            in_specs=[pl.BlockSpec((1, GATHER_WIN), lambda i: (0, i))],
            out_specs=[pl.BlockSpec((GATHER_WIN, DIM), lambda i: (i, 0))],
            core_axis_name=("core", "subcore"),
            dimension_semantics=(pltpu.PARALLEL,),
        )(i_hbm, o_hbm)

    return knl(codebook, idx_row)


def kernel(x, codebook):
    x_flat = x.reshape(-1, DIM)
    idx_blocks, discrete = _tc_argmin_onehot(x_flat, codebook)
    idx_row = idx_blocks.reshape(1, M_TOTAL)
    quantized = _sc_gather(codebook, idx_row).reshape(x.shape)
    return (discrete, quantized)


# speedup vs baseline: 1.7061x; 1.0995x over previous
import jax
import jax.numpy as jnp
from jax.experimental import pallas as pl
from jax.experimental.pallas import tpu as pltpu
from jax.experimental.pallas import tpu_sc as plsc

N_EMB = 8192
DIM = 256
M_TOTAL = 8192
M_BLK = 256
NB = M_TOTAL // M_BLK
N_CHUNK = 128
NC = N_EMB // N_CHUNK

GATHER_WIN = 128


def _tc_body(x_ref, cb_ref, idx_ref, oh_ref, csq_ref, cbbf_ref, previdx_ref):
    i = pl.program_id(0)

    @pl.when(i == 0)
    def _init():
        cb = cb_ref[...]
        ones = jnp.ones((1, DIM), jnp.float32)
        csq_ref[...] = jax.lax.dot_general(
            ones, cb * cb, (((1,), (1,)), ((), ())),
            precision=jax.lax.Precision.HIGHEST,
            preferred_element_type=jnp.float32)
        cbbf_ref[...] = cb.astype(jnp.bfloat16)

    ids = jax.lax.broadcasted_iota(jnp.int32, (M_BLK, N_EMB), 1)
    oh_ref[...] = (ids == previdx_ref[...]).astype(jnp.float32)

    @pl.when(i < NB)
    def _compute():
        xb = x_ref[...]
        xsq = jnp.sum(xb * xb, axis=1, keepdims=True)
        xbf = (xb * -2.0).astype(jnp.bfloat16)
        iota_w = jax.lax.broadcasted_iota(jnp.int32, (M_BLK, N_CHUNK), 1)
        minv = None
        argv = None
        for k in range(NC):
            cbk = cbbf_ref[pl.ds(k * N_CHUNK, N_CHUNK), :]
            dot = jax.lax.dot_general(
                xbf, cbk, (((1,), (1,)), ((), ())),
                preferred_element_type=jnp.float32)
            csqk = csq_ref[:, pl.ds(k * N_CHUNK, N_CHUNK)]
            d = (xsq + dot) + csqk
            col = iota_w + (k * N_CHUNK)
            if minv is None:
                minv, argv = d, col
            else:
                upd = d < minv
                minv = jnp.where(upd, d, minv)
                argv = jnp.where(upd, col, argv)
        mn = jnp.min(minv, axis=1, keepdims=True)
        idx = jnp.min(jnp.where(minv == mn, argv, jnp.int32(N_EMB)),
                      axis=1, keepdims=True)
        idx_ref[0, :, :] = idx
        previdx_ref[...] = idx

    @pl.when(i == NB)
    def _tail():
        idx_ref[0, :, :] = previdx_ref[...]


def _tc_argmin_onehot(x_flat, codebook, interpret=False):
    grid = (NB + 1,)
    out_shapes = (
        jax.ShapeDtypeStruct((NB, M_BLK, 1), jnp.int32),
        jax.ShapeDtypeStruct((M_TOTAL, N_EMB), jnp.float32),
    )
    return pl.pallas_call(
        _tc_body,
        grid=grid,
        in_specs=[
            pl.BlockSpec((M_BLK, DIM), lambda i: (jnp.minimum(i, NB - 1), 0)),
            pl.BlockSpec((N_EMB, DIM), lambda i: (0, 0)),
        ],
        out_specs=(
            pl.BlockSpec((1, M_BLK, 1), lambda i: (jnp.minimum(i, NB - 1), 0, 0)),
            pl.BlockSpec((M_BLK, N_EMB), lambda i: (jnp.maximum(i - 1, 0), 0)),
        ),
        out_shape=out_shapes,
        scratch_shapes=[
            pltpu.VMEM((1, N_EMB), jnp.float32),
            pltpu.VMEM((N_EMB, DIM), jnp.bfloat16),
            pltpu.VMEM((M_BLK, 1), jnp.int32),
        ],
        interpret=interpret,
    )(x_flat, codebook)


def _sc_gather(codebook, idx_row):
    mesh = plsc.VectorSubcoreMesh(core_axis_name="core",
                                  subcore_axis_name="subcore")

    @pl.kernel(out_type=jax.ShapeDtypeStruct((M_TOTAL, DIM), jnp.float32),
               mesh=mesh)
    def knl(cb_hbm, i_hbm, o_hbm):
        def body(i_vmem, o_vmem):
            pltpu.sync_copy(cb_hbm.at[i_vmem.at[0]], o_vmem)

        pltpu.emit_pipeline(
            body,
            grid=(M_TOTAL // GATHER_WIN,),
            in_specs=[pl.BlockSpec((1, GATHER_WIN), lambda i: (0, i))],
            out_specs=[pl.BlockSpec((GATHER_WIN, DIM), lambda i: (i, 0))],
            core_axis_name=("core", "subcore"),
            dimension_semantics=(pltpu.PARALLEL,),
        )(i_hbm, o_hbm)

    return knl(codebook, idx_row)


def kernel(x, codebook):
    x_flat = x.reshape(-1, DIM)
    idx_blocks, discrete = _tc_argmin_onehot(x_flat, codebook)
    idx_row = idx_blocks.reshape(1, M_TOTAL)
    quantized = _sc_gather(codebook, idx_row).reshape(x.shape)
    return (discrete, quantized)


# speedup vs baseline: 1.7758x; 1.0409x over previous
import jax
import jax.numpy as jnp
from jax.experimental import pallas as pl
from jax.experimental.pallas import tpu as pltpu
from jax.experimental.pallas import tpu_sc as plsc

N_EMB = 8192
DIM = 256
M_TOTAL = 8192
M_BLK = 256
NB = M_TOTAL // M_BLK
N_CHUNK = 256
NC = N_EMB // N_CHUNK

GATHER_WIN = 128


def _tc_body(x_ref, cb_ref, idx_ref, oh_ref, csq_ref, cbbf_ref, previdx_ref):
    i = pl.program_id(0)

    @pl.when(i == 0)
    def _init():
        cb = cb_ref[...]
        ones = jnp.ones((1, DIM), jnp.float32)
        csq_ref[...] = jax.lax.dot_general(
            ones, cb * cb, (((1,), (1,)), ((), ())),
            precision=jax.lax.Precision.HIGHEST,
            preferred_element_type=jnp.float32)
        cbbf_ref[...] = cb.astype(jnp.bfloat16)

    ids = jax.lax.broadcasted_iota(jnp.int32, (M_BLK, N_EMB), 1)
    oh_ref[...] = (ids == previdx_ref[...]).astype(jnp.float32)

    @pl.when(i < NB)
    def _compute():
        xb = x_ref[...]
        xsq = jnp.sum(xb * xb, axis=1, keepdims=True)
        xbf = (xb * -2.0).astype(jnp.bfloat16)
        minv = None
        argv = None
        for k in range(NC):
            cbk = cbbf_ref[pl.ds(k * N_CHUNK, N_CHUNK), :]
            dot = jax.lax.dot_general(
                xbf, cbk, (((1,), (1,)), ((), ())),
                preferred_element_type=jnp.float32)
            csqk = csq_ref[:, pl.ds(k * N_CHUNK, N_CHUNK)]
            d = (xsq + dot) + csqk
            if minv is None:
                minv = d
                argv = jnp.zeros((M_BLK, N_CHUNK), jnp.int32)
            else:
                upd = d < minv
                minv = jnp.where(upd, d, minv)
                argv = jnp.where(upd, jnp.int32(k), argv)
        iota_w = jax.lax.broadcasted_iota(jnp.int32, (M_BLK, N_CHUNK), 1)
        col = argv * N_CHUNK + iota_w
        mn = jnp.min(minv, axis=1, keepdims=True)
        idx = jnp.min(jnp.where(minv == mn, col, jnp.int32(N_EMB)),
                      axis=1, keepdims=True)
        idx_ref[0, :, :] = idx
        previdx_ref[...] = idx

    @pl.when(i == NB)
    def _tail():
        idx_ref[0, :, :] = previdx_ref[...]


def _tc_argmin_onehot(x_flat, codebook, interpret=False):
    grid = (NB + 1,)
    out_shapes = (
        jax.ShapeDtypeStruct((NB, M_BLK, 1), jnp.int32),
        jax.ShapeDtypeStruct((M_TOTAL, N_EMB), jnp.float32),
    )
    return pl.pallas_call(
        _tc_body,
        grid=grid,
        in_specs=[
            pl.BlockSpec((M_BLK, DIM), lambda i: (jnp.minimum(i, NB - 1), 0)),
            pl.BlockSpec((N_EMB, DIM), lambda i: (0, 0)),
        ],
        out_specs=(
            pl.BlockSpec((1, M_BLK, 1), lambda i: (jnp.minimum(i, NB - 1), 0, 0)),
            pl.BlockSpec((M_BLK, N_EMB), lambda i: (jnp.maximum(i - 1, 0), 0)),
        ),
        out_shape=out_shapes,
        scratch_shapes=[
            pltpu.VMEM((1, N_EMB), jnp.float32),
            pltpu.VMEM((N_EMB, DIM), jnp.bfloat16),
            pltpu.VMEM((M_BLK, 1), jnp.int32),
        ],
        interpret=interpret,
    )(x_flat, codebook)


def _sc_gather(codebook, idx_row):
    mesh = plsc.VectorSubcoreMesh(core_axis_name="core",
                                  subcore_axis_name="subcore")

    @pl.kernel(out_type=jax.ShapeDtypeStruct((M_TOTAL, DIM), jnp.float32),
               mesh=mesh)
    def knl(cb_hbm, i_hbm, o_hbm):
        def body(i_vmem, o_vmem):
            pltpu.sync_copy(cb_hbm.at[i_vmem.at[0]], o_vmem)

        pltpu.emit_pipeline(
            body,
            grid=(M_TOTAL // GATHER_WIN,),
            in_specs=[pl.BlockSpec((1, GATHER_WIN), lambda i: (0, i))],
            out_specs=[pl.BlockSpec((GATHER_WIN, DIM), lambda i: (i, 0))],
            core_axis_name=("core", "subcore"),
            dimension_semantics=(pltpu.PARALLEL,),
        )(i_hbm, o_hbm)

    return knl(codebook, idx_row)


def kernel(x, codebook):
    x_flat = x.reshape(-1, DIM)
    idx_blocks, discrete = _tc_argmin_onehot(x_flat, codebook)
    idx_row = idx_blocks.reshape(1, M_TOTAL)
    quantized = _sc_gather(codebook, idx_row).reshape(x.shape)
    return (discrete, quantized)


# speedup vs baseline: 1.8563x; 1.0453x over previous
import jax
import jax.numpy as jnp
from jax.experimental import pallas as pl
from jax.experimental.pallas import tpu as pltpu
from jax.experimental.pallas import tpu_sc as plsc

N_EMB = 8192
DIM = 256
M_TOTAL = 8192
M_BLK = 512
NB = M_TOTAL // M_BLK
N_CHUNK = 256
NC = N_EMB // N_CHUNK

GATHER_WIN = 128


def _tc_body(x_ref, cb_ref, idx_ref, oh_ref, csq_ref, cbbf_ref, previdx_ref):
    i = pl.program_id(0)

    @pl.when(i == 0)
    def _init():
        cb = cb_ref[...]
        ones = jnp.ones((1, DIM), jnp.float32)
        csq_ref[...] = jax.lax.dot_general(
            ones, cb * cb, (((1,), (1,)), ((), ())),
            precision=jax.lax.Precision.HIGHEST,
            preferred_element_type=jnp.float32)
        cbbf_ref[...] = cb.astype(jnp.bfloat16)

    ids = jax.lax.broadcasted_iota(jnp.int32, (M_BLK, N_EMB), 1)
    oh_ref[...] = (ids == previdx_ref[...]).astype(jnp.float32)

    @pl.when(i < NB)
    def _compute():
        xb = x_ref[...]
        xsq = jnp.sum(xb * xb, axis=1, keepdims=True)
        xbf = (xb * -2.0).astype(jnp.bfloat16)
        half = N_CHUNK // 2
        minv = None
        argv = None
        for k in range(NC):
            cbk = cbbf_ref[pl.ds(k * N_CHUNK, N_CHUNK), :]
            dot = jax.lax.dot_general(
                xbf, cbk, (((1,), (1,)), ((), ())),
                preferred_element_type=jnp.float32)
            csqk = csq_ref[:, pl.ds(k * N_CHUNK, N_CHUNK)]
            d = (xsq + dot) + csqk
            dl = d[:, :half]
            dr = d[:, half:]
            fr = dr < dl
            df = jnp.where(fr, dr, dl)
            cid = jnp.where(fr, jnp.int32(2 * k + 1), jnp.int32(2 * k))
            if minv is None:
                minv, argv = df, cid
            else:
                upd = df < minv
                minv = jnp.where(upd, df, minv)
                argv = jnp.where(upd, cid, argv)
        iota_w = jax.lax.broadcasted_iota(jnp.int32, (M_BLK, half), 1)
        col = argv * half + iota_w
        mn = jnp.min(minv, axis=1, keepdims=True)
        idx = jnp.min(jnp.where(minv == mn, col, jnp.int32(N_EMB)),
                      axis=1, keepdims=True)
        idx_ref[0, :, :] = idx
        previdx_ref[...] = idx

    @pl.when(i == NB)
    def _tail():
        idx_ref[0, :, :] = previdx_ref[...]


def _tc_argmin_onehot(x_flat, codebook, interpret=False):
    grid = (NB + 1,)
    out_shapes = (
        jax.ShapeDtypeStruct((NB, M_BLK, 1), jnp.int32),
        jax.ShapeDtypeStruct((M_TOTAL, N_EMB), jnp.float32),
    )
    return pl.pallas_call(
        _tc_body,
        grid=grid,
        in_specs=[
            pl.BlockSpec((M_BLK, DIM), lambda i: (jnp.minimum(i, NB - 1), 0)),
            pl.BlockSpec((N_EMB, DIM), lambda i: (0, 0)),
        ],
        out_specs=(
            pl.BlockSpec((1, M_BLK, 1), lambda i: (jnp.minimum(i, NB - 1), 0, 0)),
            pl.BlockSpec((M_BLK, N_EMB), lambda i: (jnp.maximum(i - 1, 0), 0)),
        ),
        out_shape=out_shapes,
        scratch_shapes=[
            pltpu.VMEM((1, N_EMB), jnp.float32),
            pltpu.VMEM((N_EMB, DIM), jnp.bfloat16),
            pltpu.VMEM((M_BLK, 1), jnp.int32),
        ],
        interpret=interpret,
    )(x_flat, codebook)


def _sc_gather(codebook, idx_row):
    mesh = plsc.VectorSubcoreMesh(core_axis_name="core",
                                  subcore_axis_name="subcore")

    @pl.kernel(out_type=jax.ShapeDtypeStruct((M_TOTAL, DIM), jnp.float32),
               mesh=mesh)
    def knl(cb_hbm, i_hbm, o_hbm):
        def body(i_vmem, o_vmem):
            pltpu.sync_copy(cb_hbm.at[i_vmem.at[0]], o_vmem)

        pltpu.emit_pipeline(
            body,
            grid=(M_TOTAL // GATHER_WIN,),
            in_specs=[pl.BlockSpec((1, GATHER_WIN), lambda i: (0, i))],
            out_specs=[pl.BlockSpec((GATHER_WIN, DIM), lambda i: (i, 0))],
            core_axis_name=("core", "subcore"),
            dimension_semantics=(pltpu.PARALLEL,),
        )(i_hbm, o_hbm)

    return knl(codebook, idx_row)


def kernel(x, codebook):
    x_flat = x.reshape(-1, DIM)
    idx_blocks, discrete = _tc_argmin_onehot(x_flat, codebook)
    idx_row = idx_blocks.reshape(1, M_TOTAL)
    quantized = _sc_gather(codebook, idx_row).reshape(x.shape)
    return (discrete, quantized)
